# baseline (speedup 1.0000x reference)
import functools

import jax
import jax.numpy as jnp
from jax import lax
from jax.experimental import pallas as pl
from jax.experimental.pallas import tpu as pltpu
from jax.experimental.pallas import tpu_sc as plsc

LANES = 16
GROUP = 32
NB = 10
NC, NS = 2, 16
NW = NC * NS


def _combo_body(seg_ref, pos_ref, out_ref):
    S = seg_ref.shape[0]
    L = pos_ref.shape[0]
    for s in range(S):
        out_ref[s * L:(s + 1) * L, :] = pos_ref[...] + seg_ref[s:s + 1, :]


def _build_combo(segment_table, position_table):
    S, H = segment_table.shape
    L = position_table.shape[0]
    return pl.pallas_call(
        _combo_body,
        out_shape=jax.ShapeDtypeStruct((S * L, H), jnp.float32),
    )(segment_table, position_table)


def _sc_embed(xf, segf, token_table, combo, L, H, gpw):
    N = NW * gpw * GROUP
    n_outer = gpw // NB
    mesh = plsc.VectorSubcoreMesh(core_axis_name="c", subcore_axis_name="s")

    @functools.partial(
        pl.kernel,
        mesh=mesh,
        out_type=jax.ShapeDtypeStruct((N, H), jnp.float32),
        scratch_types=[
            pltpu.VMEM((gpw, GROUP), jnp.int32),
            pltpu.VMEM((gpw, GROUP), jnp.int32),
        ] + [pltpu.VMEM((GROUP, H), jnp.float32)] * NB
          + [pltpu.SemaphoreType.DMA] * NB,
    )
    def k(xf_hbm, segf_hbm, tok_hbm, combo_hbm, out_hbm, idx_v, cidx_v, *rest):
        bufs = rest[:NB]
        sems = rest[NB:]
        wid = lax.axis_index("s") * NC + lax.axis_index("c")
        gbase = wid * gpw
        pltpu.sync_copy(xf_hbm.at[wid], idx_v)
        pltpu.sync_copy(segf_hbm.at[wid], cidx_v)

        iota = lax.iota(jnp.int32, LANES)

        def cidx_body(r, carry):
            row0 = (gbase + r) * GROUP
            for j in range(GROUP // LANES):
                sl = pl.ds(j * LANES, LANES)
                p = lax.rem(row0 + j * LANES + iota, L)
                cidx_v[r, sl] = cidx_v[r, sl] * L + p
            return carry

        lax.fori_loop(0, gpw, cidx_body, 0)

        def issue_c(b, g):
            pltpu.async_copy(combo_hbm.at[cidx_v.at[g]], bufs[b], sems[b])

        def issue_t(b, g):
            pltpu.async_copy(tok_hbm.at[idx_v.at[g]], bufs[b], sems[b], add=True)

        def issue_s(b, g):
            pltpu.async_copy(bufs[b], out_hbm.at[pl.ds((gbase + g) * GROUP, GROUP)],
                             sems[b])

        def wait_gather(b):
            pltpu.make_async_copy(combo_hbm.at[pl.ds(0, GROUP)], bufs[b],
                                  sems[b]).wait()

        def wait_store(b):
            pltpu.make_async_copy(bufs[b], out_hbm.at[pl.ds(0, GROUP)],
                                  sems[b]).wait()

        def slot(b, g, first_ring):
            if not first_ring:
                wait_store(b)
            issue_c(b, g)
            if (not first_ring) or b >= 1:
                b1 = (b - 1) % NB
                wait_gather(b1)
                issue_t(b1, g - 1)
            if (not first_ring) or b >= 2:
                b2 = (b - 2) % NB
                wait_gather(b2)
                issue_s(b2, g - 2)

        for b in range(NB):
            slot(b, b, True)

        def outer_body(o, carry):
            g0 = o * NB
            for b in range(NB):
                slot(b, g0 + b, False)
            return carry

        lax.fori_loop(1, n_outer, outer_body, 0)

        b_last = (gpw - 1) % NB
        b_prev = (gpw - 2) % NB
        wait_gather(b_last)
        issue_t(b_last, gpw - 1)
        wait_gather(b_prev)
        issue_s(b_prev, gpw - 2)
        wait_gather(b_last)
        issue_s(b_last, gpw - 1)
        for b in range(NB):
            wait_store(b)

    return k(xf, segf, token_table, combo)


def kernel(x, segment_ids, token_table, segment_table, position_table):
    B, L = x.shape
    V, H = token_table.shape
    N = B * L
    assert N % (NW * GROUP) == 0
    gpw = N // (NW * GROUP)
    assert gpw % NB == 0

    combo = _build_combo(segment_table, position_table)
    xf = x.reshape(NW, gpw, GROUP)
    segf = segment_ids.reshape(NW, gpw, GROUP)
    out = _sc_embed(xf, segf, token_table, combo, L, H, gpw)
    return out.reshape(B, L, H)


# speedup vs baseline: 1.0085x; 1.0085x over previous
import functools

import jax
import jax.numpy as jnp
from jax import lax
from jax.experimental import pallas as pl
from jax.experimental.pallas import tpu as pltpu
from jax.experimental.pallas import tpu_sc as plsc

LANES = 16
GROUP = 80
NB = 10
NC, NS = 2, 16
NW = NC * NS


def _combo_body(seg_ref, pos_ref, out_ref):
    S = seg_ref.shape[0]
    L = pos_ref.shape[0]
    for s in range(S):
        out_ref[s * L:(s + 1) * L, :] = pos_ref[...] + seg_ref[s:s + 1, :]


def _build_combo(segment_table, position_table):
    S, H = segment_table.shape
    L = position_table.shape[0]
    return pl.pallas_call(
        _combo_body,
        out_shape=jax.ShapeDtypeStruct((S * L, H), jnp.float32),
    )(segment_table, position_table)


def _sc_embed(xf, segf, token_table, combo, L, H, gpw):
    N = NW * gpw * GROUP
    n_outer = gpw // NB
    mesh = plsc.VectorSubcoreMesh(core_axis_name="c", subcore_axis_name="s")

    @functools.partial(
        pl.kernel,
        mesh=mesh,
        out_type=jax.ShapeDtypeStruct((N, H), jnp.float32),
        scratch_types=[
            pltpu.VMEM((gpw, GROUP), jnp.int32),
            pltpu.VMEM((gpw, GROUP), jnp.int32),
        ] + [pltpu.VMEM((GROUP, H), jnp.float32)] * NB
          + [pltpu.SemaphoreType.DMA] * NB,
    )
    def k(xf_hbm, segf_hbm, tok_hbm, combo_hbm, out_hbm, idx_v, cidx_v, *rest):
        bufs = rest[:NB]
        sems = rest[NB:]
        wid = lax.axis_index("s") * NC + lax.axis_index("c")
        gbase = wid * gpw
        pltpu.sync_copy(xf_hbm.at[wid], idx_v)
        pltpu.sync_copy(segf_hbm.at[wid], cidx_v)

        iota = lax.iota(jnp.int32, LANES)

        def cidx_body(r, carry):
            row0 = (gbase + r) * GROUP
            for j in range(GROUP // LANES):
                sl = pl.ds(j * LANES, LANES)
                p = lax.rem(row0 + j * LANES + iota, L)
                cidx_v[r, sl] = cidx_v[r, sl] * L + p
            return carry

        lax.fori_loop(0, gpw, cidx_body, 0)

        def issue_c(b, g):
            pltpu.async_copy(combo_hbm.at[cidx_v.at[g]], bufs[b], sems[b])

        def issue_t(b, g):
            pltpu.async_copy(tok_hbm.at[idx_v.at[g]], bufs[b], sems[b], add=True)

        def issue_s(b, g):
            pltpu.async_copy(bufs[b], out_hbm.at[pl.ds((gbase + g) * GROUP, GROUP)],
                             sems[b])

        def wait_gather(b):
            pltpu.make_async_copy(combo_hbm.at[pl.ds(0, GROUP)], bufs[b],
                                  sems[b]).wait()

        def wait_store(b):
            pltpu.make_async_copy(bufs[b], out_hbm.at[pl.ds(0, GROUP)],
                                  sems[b]).wait()

        def slot(b, g, first_ring):
            if not first_ring:
                wait_store(b)
            issue_c(b, g)
            if (not first_ring) or b >= 1:
                b1 = (b - 1) % NB
                wait_gather(b1)
                issue_t(b1, g - 1)
            if (not first_ring) or b >= 2:
                b2 = (b - 2) % NB
                wait_gather(b2)
                issue_s(b2, g - 2)

        for b in range(NB):
            slot(b, b, True)

        def outer_body(o, carry):
            g0 = o * NB
            for b in range(NB):
                slot(b, g0 + b, False)
            return carry

        lax.fori_loop(1, n_outer, outer_body, 0)

        b_last = (gpw - 1) % NB
        b_prev = (gpw - 2) % NB
        wait_gather(b_last)
        issue_t(b_last, gpw - 1)
        wait_gather(b_prev)
        issue_s(b_prev, gpw - 2)
        wait_gather(b_last)
        issue_s(b_last, gpw - 1)
        for b in range(NB):
            wait_store(b)

    return k(xf, segf, token_table, combo)


def kernel(x, segment_ids, token_table, segment_table, position_table):
    B, L = x.shape
    V, H = token_table.shape
    N = B * L
    assert N % (NW * GROUP) == 0
    gpw = N // (NW * GROUP)
    assert gpw % NB == 0

    combo = _build_combo(segment_table, position_table)
    xf = x.reshape(NW, gpw, GROUP)
    segf = segment_ids.reshape(NW, gpw, GROUP)
    out = _sc_embed(xf, segf, token_table, combo, L, H, gpw)
    return out.reshape(B, L, H)


# speedup vs baseline: 1.1765x; 1.1667x over previous
import functools

import jax
import jax.numpy as jnp
from jax import lax
from jax.experimental import pallas as pl
from jax.experimental.pallas import tpu as pltpu
from jax.experimental.pallas import tpu_sc as plsc

LANES = 16
GROUP = 64
NB = 5
NC, NS = 2, 16
NW = NC * NS


def _combo_body(seg_ref, pos_ref, out_ref):
    S = seg_ref.shape[0]
    L = pos_ref.shape[0]
    for s in range(S):
        out_ref[s * L:(s + 1) * L, :] = pos_ref[...] + seg_ref[s:s + 1, :]


def _build_combo(segment_table, position_table):
    S, H = segment_table.shape
    L = position_table.shape[0]
    return pl.pallas_call(
        _combo_body,
        out_shape=jax.ShapeDtypeStruct((S * L, H), jnp.float32),
    )(segment_table, position_table)


def _sc_embed(xf, segf, token_table, combo, L, H, gpw):
    N = NW * gpw * GROUP
    n_outer = gpw // NB
    mesh = plsc.VectorSubcoreMesh(core_axis_name="c", subcore_axis_name="s")

    @functools.partial(
        pl.kernel,
        mesh=mesh,
        out_type=jax.ShapeDtypeStruct((N, H), jnp.float32),
        scratch_types=[
            pltpu.VMEM((gpw, GROUP), jnp.int32),
            pltpu.VMEM((gpw, GROUP), jnp.int32),
        ] + [pltpu.VMEM((GROUP, H), jnp.float32)] * NB
          + [pltpu.SemaphoreType.DMA] * NB,
    )
    def k(xf_hbm, segf_hbm, tok_hbm, combo_hbm, out_hbm, idx_v, cidx_v, *rest):
        bufs = rest[:NB]
        sems = rest[NB:]
        wid = lax.axis_index("s") * NC + lax.axis_index("c")
        gbase = wid * gpw
        pltpu.sync_copy(xf_hbm.at[wid], idx_v)
        pltpu.sync_copy(segf_hbm.at[wid], cidx_v)

        iota = lax.iota(jnp.int32, LANES)

        def cidx_body(r, carry):
            row0 = (gbase + r) * GROUP
            for j in range(GROUP // LANES):
                sl = pl.ds(j * LANES, LANES)
                p = lax.rem(row0 + j * LANES + iota, L)
                cidx_v[r, sl] = cidx_v[r, sl] * L + p
            return carry

        lax.fori_loop(0, gpw, cidx_body, 0)

        def issue_c(b, g):
            pltpu.async_copy(combo_hbm.at[cidx_v.at[g]], bufs[b], sems[b])

        def issue_t(b, g):
            pltpu.async_copy(tok_hbm.at[idx_v.at[g]], bufs[b], sems[b], add=True)

        def issue_s(b, g):
            pltpu.async_copy(bufs[b], out_hbm.at[pl.ds((gbase + g) * GROUP, GROUP)],
                             sems[b])

        def wait_gather(b):
            pltpu.make_async_copy(combo_hbm.at[pl.ds(0, GROUP)], bufs[b],
                                  sems[b]).wait()

        def wait_store(b):
            pltpu.make_async_copy(bufs[b], out_hbm.at[pl.ds(0, GROUP)],
                                  sems[b]).wait()

        def slot(b, g, first_ring):
            if not first_ring:
                wait_store(b)
            issue_c(b, g)
            if (not first_ring) or b >= 1:
                b1 = (b - 1) % NB
                wait_gather(b1)
                issue_t(b1, g - 1)
            if (not first_ring) or b >= 2:
                b2 = (b - 2) % NB
                wait_gather(b2)
                issue_s(b2, g - 2)

        for b in range(NB):
            slot(b, b, True)

        def outer_body(o, carry):
            g0 = o * NB
            for b in range(NB):
                slot(b, g0 + b, False)
            return carry

        lax.fori_loop(1, n_outer, outer_body, 0)

        b_last = (gpw - 1) % NB
        b_prev = (gpw - 2) % NB
        wait_gather(b_last)
        issue_t(b_last, gpw - 1)
        wait_gather(b_prev)
        issue_s(b_prev, gpw - 2)
        wait_gather(b_last)
        issue_s(b_last, gpw - 1)
        for b in range(NB):
            wait_store(b)

    return k(xf, segf, token_table, combo)


def kernel(x, segment_ids, token_table, segment_table, position_table):
    B, L = x.shape
    V, H = token_table.shape
    N = B * L
    assert N % (NW * GROUP) == 0
    gpw = N // (NW * GROUP)
    assert gpw % NB == 0

    combo = _build_combo(segment_table, position_table)
    xf = x.reshape(NW, gpw, GROUP)
    segf = segment_ids.reshape(NW, gpw, GROUP)
    out = _sc_embed(xf, segf, token_table, combo, L, H, gpw)
    return out.reshape(B, L, H)
